# baseline (speedup 1.0000x reference)
import jax
import jax.numpy as jnp
from jax.experimental import pallas as pl

_N_E = 1024
_E_DIM = 256
_BETA = 0.25
_B = 8
_TOK = 32 * 32
_N_TOK = _B * _TOK


def _vq_kernel(x_ref, emb_ref, q_ref, idx_ref, loss_ref, counts_ref, perp_ref):
    i = pl.program_id(0)
    x = x_ref[0]
    emb = emb_ref[...]
    dots = jax.lax.dot_general(x, emb, (((0,), (1,)), ((), ())),
                               preferred_element_type=jnp.float32)
    z2 = jnp.sum(x * x, axis=0, keepdims=True)
    z2col = jnp.transpose(z2)
    e2 = jnp.sum(emb * emb, axis=1)
    d = (z2col + e2[None, :]) - 2.0 * dots
    dmin = jnp.min(d, axis=1, keepdims=True)
    cols = jax.lax.broadcasted_iota(jnp.int32, d.shape, 1)
    idx = jnp.min(jnp.where(d <= dmin, cols, _N_E), axis=1, keepdims=True)
    idx_row = jnp.transpose(idx)
    code_iota = jax.lax.broadcasted_iota(jnp.int32, (_N_E, _TOK), 0)
    onehot = (code_iota == idx_row).astype(jnp.float32)
    zq = jax.lax.dot_general(emb, onehot, (((0,), (0,)), ((), ())),
                             preferred_element_type=jnp.float32)
    q_ref[0] = zq
    idx_ref[0] = idx_row

    diff = zq - x
    sse_part = jnp.sum(diff * diff)
    counts_part = jnp.sum(onehot, axis=1, keepdims=True)

    @pl.when(i == 0)
    def _init():
        loss_ref[...] = jnp.zeros_like(loss_ref)
        counts_ref[...] = jnp.zeros_like(counts_ref)
        perp_ref[...] = jnp.zeros_like(perp_ref)

    loss_ref[...] += jnp.full((1, 1), sse_part, jnp.float32)
    counts_ref[...] += counts_part

    @pl.when(i == _B - 1)
    def _finish():
        probs = counts_ref[...] / _N_TOK
        ent = jnp.sum(probs * jnp.log(probs + 1e-10))
        perp_ref[...] = jnp.full((1, 1), jnp.exp(-ent), jnp.float32)
        loss_ref[...] = loss_ref[...] * ((1.0 + _BETA) / (_N_TOK * _E_DIM))


def kernel(inputs, embedding):
    b, c, h, w = inputs.shape
    x = inputs.reshape(b, c, h * w)
    q, idx, loss, _counts, perp = pl.pallas_call(
        _vq_kernel,
        grid=(_B,),
        in_specs=[
            pl.BlockSpec((1, _E_DIM, _TOK), lambda i: (i, 0, 0)),
            pl.BlockSpec((_N_E, _E_DIM), lambda i: (0, 0)),
        ],
        out_specs=[
            pl.BlockSpec((1, _E_DIM, _TOK), lambda i: (i, 0, 0)),
            pl.BlockSpec((1, 1, _TOK), lambda i: (i, 0, 0)),
            pl.BlockSpec((1, 1), lambda i: (0, 0)),
            pl.BlockSpec((_N_E, 1), lambda i: (0, 0)),
            pl.BlockSpec((1, 1), lambda i: (0, 0)),
        ],
        out_shape=[
            jax.ShapeDtypeStruct((_B, _E_DIM, _TOK), jnp.float32),
            jax.ShapeDtypeStruct((_B, 1, _TOK), jnp.int32),
            jax.ShapeDtypeStruct((1, 1), jnp.float32),
            jax.ShapeDtypeStruct((_N_E, 1), jnp.float32),
            jax.ShapeDtypeStruct((1, 1), jnp.float32),
        ],
    )(x, embedding)
    quantized = q.reshape(b, c, h, w)
    encoding_indices = idx.reshape(b, h, w)
    return (loss[0, 0], quantized, perp[0, 0], encoding_indices)
